# baseline (speedup 1.0000x reference)
import jax
import jax.numpy as jnp
from jax.experimental import pallas as pl
from jax.experimental.pallas import tpu as pltpu

_B, _N, _DIN, _H, _DOUT = 4, 2048, 128, 64, 16
_F = _N * 2 * _DOUT
_Q = 4
_RS = 4
_C = _N // (_Q * _RS)
_OR = _N // _RS
_FQ = _F // _Q


def _gcn_body(x_ref, s0, s1, s2, s3, w1_ref, w2_ref, out_ref,
              slab, h1slab, xw_ref, hw_ref):
    b = pl.program_id(0)
    r = pl.program_id(1)
    srcs = [s0, s1, s2, s3]
    sb = jax.lax.rem(b, 2)
    pb = jax.lax.rem(b + 1, 2)

    @pl.when(jnp.logical_and(b < _B, r == 0))
    def _():
        xw_ref[...] = jnp.dot(x_ref[0], w1_ref[...],
                              preferred_element_type=jnp.float32)

    @pl.when(b < _B)
    def _():
        for q in range(_Q):
            off = (q * _RS) * _C
            slab[sb, pl.ds(off + r * _C, _C), :] = srcs[q][0]
            h1slab[sb, pl.ds(off + r * _C, _C), :] = jnp.maximum(
                jnp.dot(srcs[q][0], xw_ref[...],
                        preferred_element_type=jnp.float32), 0.0)

    @pl.when(b > 0)
    def _():
        @pl.when(r == 0)
        def _():
            hw_ref[...] = jnp.dot(h1slab[pb], w2_ref[...],
                                  preferred_element_type=jnp.float32)

        out_ref[0] = jnp.maximum(
            jnp.dot(slab[pb, pl.ds(r * _OR, _OR), :], hw_ref[...],
                    preferred_element_type=jnp.float32), 0.0)


def _readout_body(f_ref, w0, w1, w2, w3, br1_ref, wr2_ref, br2_ref, out_ref):
    ws = [w0, w1, w2, w3]
    o1 = jnp.zeros((_B, 64), jnp.float32)
    for q in range(_Q):
        o1 = o1 + jnp.dot(f_ref[:, q * _FQ:(q + 1) * _FQ], ws[q][...],
                          preferred_element_type=jnp.float32)
    o1 = jnp.maximum(o1 + br1_ref[...], 0.0)
    o = jnp.dot(o1, wr2_ref[...], preferred_element_type=jnp.float32)
    o = o + br2_ref[...]
    m = jnp.max(o, axis=-1, keepdims=True)
    lse = m + jnp.log(jnp.sum(jnp.exp(o - m), axis=-1, keepdims=True))
    out_ref[...] = o - lse


def _s_spec(q):
    def idx(b, r, q=q):
        bb = jnp.minimum(b, _B - 1)
        rr = jnp.where(b >= _B, _RS - 1, r)
        return (bb, q * _RS + rr, 0)
    return pl.BlockSpec((1, _C, _N), idx)


@jax.jit
def kernel(x, support, W1, W2, Wr1, br1, Wr2, br2):
    h2 = pl.pallas_call(
        _gcn_body,
        grid=(_B + 1, _RS),
        in_specs=[
            pl.BlockSpec((1, _N, _DIN),
                         lambda b, r: (jnp.minimum(b, _B - 1), 0, 0)),
            _s_spec(0), _s_spec(1), _s_spec(2), _s_spec(3),
            pl.BlockSpec((_DIN, _H), lambda b, r: (0, 0)),
            pl.BlockSpec((_H, 2 * _DOUT), lambda b, r: (0, 0)),
        ],
        out_specs=pl.BlockSpec(
            (1, _OR, 2 * _DOUT),
            lambda b, r: (jnp.maximum(b - 1, 0), r, 0)),
        out_shape=jax.ShapeDtypeStruct((_B, _N, 2 * _DOUT), jnp.float32),
        scratch_shapes=[
            pltpu.VMEM((2, _N, _N), jnp.float32),
            pltpu.VMEM((2, _N, _H), jnp.float32),
            pltpu.VMEM((_N, _H), jnp.float32),
            pltpu.VMEM((_N, 2 * _DOUT), jnp.float32),
        ],
    )(x, support, support, support, support, W1, W2)

    f = h2.reshape(_B, _F)
    wr1_specs = [
        pl.BlockSpec((_FQ, 64), lambda g, q=q: (q, 0)) for q in range(_Q)
    ]
    out = pl.pallas_call(
        _readout_body,
        grid=(1,),
        in_specs=[pl.BlockSpec((_B, _F), lambda g: (0, 0))] + wr1_specs + [
            pl.BlockSpec((1, 64), lambda g: (0, 0)),
            pl.BlockSpec((64, _DOUT), lambda g: (0, 0)),
            pl.BlockSpec((1, _DOUT), lambda g: (0, 0)),
        ],
        out_specs=pl.BlockSpec((_B, _DOUT), lambda g: (0, 0)),
        out_shape=jax.ShapeDtypeStruct((_B, _DOUT), jnp.float32),
    )(f, Wr1, Wr1, Wr1, Wr1, br1.reshape(1, 64), Wr2, br2.reshape(1, _DOUT))
    return out
